# Initial kernel scaffold; baseline (speedup 1.0000x reference)
import jax
import jax.numpy as jnp
from jax.experimental import pallas as pl


def kernel(xyz, features, params):
    raise NotImplementedError("write your pallas kernel here")


# speedup vs baseline: 1.5037x; 1.5037x over previous
import functools

import jax
import jax.numpy as jnp
from jax import lax
from jax.experimental import pallas as pl
from jax.experimental.pallas import tpu as pltpu
from jax.experimental.pallas import tpu_sc as plsc

_B, _N, _P = 8, 4096, 1024
_BP = _B * _P
_RADII = [0.1, 0.2, 0.4]
_KS = [16, 32, 64]
_EPS = 1e-5
_CP = 48
_RT = 2048
_PT = 256
_HIGH = lax.Precision.HIGHEST


def _fps_body(xp_ref, css_ref):
    xs = xp_ref[0]
    ys = xp_ref[1]
    zs = xp_ref[2]
    iota = lax.broadcasted_iota(jnp.int32, (_B, _N), 1)
    iota128 = lax.broadcasted_iota(jnp.int32, (_B, 128), 1)

    def body(i, state):
        dists, far, cbx, cby, cbz = state
        onehot = iota == far
        cx = jnp.sum(jnp.where(onehot, xs, 0.0), axis=1, keepdims=True)
        cy = jnp.sum(jnp.where(onehot, ys, 0.0), axis=1, keepdims=True)
        cz = jnp.sum(jnp.where(onehot, zs, 0.0), axis=1, keepdims=True)
        sel = iota128 == (i & 127)
        cbx = jnp.where(sel, cx, cbx)
        cby = jnp.where(sel, cy, cby)
        cbz = jnp.where(sel, cz, cbz)

        @pl.when((i & 127) == 127)
        def _():
            b0 = pl.multiple_of(i - 127, 128)
            css_ref[0, :, pl.ds(b0, 128)] = cbx
            css_ref[1, :, pl.ds(b0, 128)] = cby
            css_ref[2, :, pl.ds(b0, 128)] = cbz

        dx = xs - cx
        dy = ys - cy
        dz = zs - cz
        d = (dx * dx + dy * dy) + dz * dz
        dists = jnp.minimum(dists, d)
        m = jnp.max(dists, axis=1, keepdims=True)
        far = jnp.min(jnp.where(dists == m, iota, _N), axis=1, keepdims=True)
        return dists, far, cbx, cby, cbz

    dists0 = jnp.full((_B, _N), 1e10, jnp.float32)
    far0 = jnp.zeros((_B, 1), jnp.int32)
    zb = jnp.zeros((_B, 128), jnp.float32)
    lax.fori_loop(0, _P, body, (dists0, far0, zb, zb, zb))


def _fps_call(xp):
    css = pl.pallas_call(
        _fps_body,
        out_shape=jax.ShapeDtypeStruct((3, _B, _P), jnp.float32),
    )(xp)
    return css


def _d2_body(xpb_ref, css_ref, out_ref):
    xs = xpb_ref[0, 0:1, :]
    ys = xpb_ref[0, 1:2, :]
    zs = xpb_ref[0, 2:3, :]
    cs = css_ref[0]
    cx = cs[:, 0:1]
    cy = cs[:, 1:2]
    cz = cs[:, 2:3]
    dx = cx - xs
    dy = cy - ys
    dz = cz - zs
    out_ref[0] = (dx * dx + dy * dy) + dz * dz


def _d2_call(xpb, css_bpc):
    return pl.pallas_call(
        _d2_body,
        grid=(_B, _P // _PT),
        in_specs=[
            pl.BlockSpec((1, 3, _N), lambda b, t: (b, 0, 0)),
            pl.BlockSpec((1, _PT, 3), lambda b, t: (b, t, 0)),
        ],
        out_specs=pl.BlockSpec((1, _PT, _N), lambda b, t: (b, t, 0)),
        out_shape=jax.ShapeDtypeStruct((_B, _P, _N), jnp.float32),
    )(xpb, css_bpc)


def _corr_body(cssf_ref, wx_ref, out_ref):
    out_ref[...] = lax.dot_general(
        cssf_ref[...], wx_ref[...],
        (((0,), (1,)), ((), ())), precision=_HIGH)


def _corr_call(cssf, wx):
    c0 = wx.shape[0]
    return pl.pallas_call(
        _corr_body,
        grid=(_BP // 512,),
        in_specs=[
            pl.BlockSpec((3, 512), lambda t: (0, t)),
            pl.BlockSpec((c0, 3), lambda t: (0, 0)),
        ],
        out_specs=pl.BlockSpec((512, c0), lambda t: (t, 0)),
        out_shape=jax.ShapeDtypeStruct((_BP, c0), jnp.float32),
    )(cssf, wx)


def _make_sc_select_gather():
    mesh = plsc.VectorSubcoreMesh(core_axis_name="c", subcore_axis_name="s")
    NC, NS = 2, 16
    NW = NC * NS
    rows_per = _BP // NW
    th = [jnp.float32(r * r) for r in _RADII]

    out_type = tuple(
        jax.ShapeDtypeStruct((_BP, k, 128), jnp.float32) for k in _KS)
    scratch = [
        pltpu.VMEM((_N,), jnp.float32),
        pltpu.VMEM((32,), jnp.int32),
        pltpu.VMEM((48,), jnp.int32),
        pltpu.VMEM((80,), jnp.int32),
        pltpu.VMEM((16,), jnp.int32),
        pltpu.VMEM((32,), jnp.int32),
        pltpu.VMEM((64,), jnp.int32),
        pltpu.VMEM((16, 128), jnp.float32),
        pltpu.VMEM((32, 128), jnp.float32),
        pltpu.VMEM((64, 128), jnp.float32),
        pltpu.SemaphoreType.DMA,
        pltpu.SemaphoreType.DMA,
    ]

    @functools.partial(
        pl.kernel, mesh=mesh, out_type=out_type, scratch_types=scratch,
        compiler_params=pltpu.CompilerParams(needs_layout_passes=False))
    def sc_kernel(d2f, tf, g1, g2, g3, bufA, ib1, ib2, ib3,
                  ix1, ix2, ix3, r1, r2, r3, semA, semG):
        def popcount_scalar(m):
            return plsc.all_reduce_population_count(m)[0]
        wid = lax.axis_index("s") * NC + lax.axis_index("c")
        base = wid * rows_per
        iota16 = lax.iota(jnp.int32, 16)
        zeros16 = jnp.zeros((16,), jnp.int32)

        def process_row(row, buf):
            b = row >> 10
            tbase = b << 12

            def body(it, st):
                c1, c2, c3 = st
                j = it * 16
                vec = buf[pl.ds(j, 16)]
                gidx = iota16 + (tbase + j)
                m1 = vec < th[0]
                m2 = vec < th[1]
                m3 = vec < th[2]
                n1 = popcount_scalar(m1)
                n2 = popcount_scalar(m2)
                n3 = popcount_scalar(m3)

                plsc.store_compressed(
                    ib1.at[pl.ds(jnp.minimum(c1, 16), 16)], gidx,
                    mask=m1 & (c1 < 16))
                plsc.store_compressed(
                    ib2.at[pl.ds(jnp.minimum(c2, 32), 16)], gidx,
                    mask=m2 & (c2 < 32))
                plsc.store_compressed(
                    ib3.at[pl.ds(jnp.minimum(c3, 64), 16)], gidx,
                    mask=m3 & (c3 < 64))

                return c1 + n1, c2 + n2, c3 + n3

            z = jnp.int32(0)
            c1, c2, c3 = lax.fori_loop(0, _N // 16, body, (z, z, z))

            def fill(ib, ix, k, cnt):
                firstv = plsc.load_gather(ib, [zeros16])
                for t in range(k // 16):
                    cur = ib[pl.ds(t * 16, 16)]
                    pos = iota16 + (t * 16)
                    ix[pl.ds(t * 16, 16)] = jnp.where(pos < cnt, cur, firstv)

            fill(ib1, ix1, 16, c1)
            fill(ib2, ix2, 32, c2)
            fill(ib3, ix3, 64, c3)

            h1 = pltpu.async_copy(tf.at[ix1], r1, semG)
            h2 = pltpu.async_copy(tf.at[ix2], r2, semG)
            h3 = pltpu.async_copy(tf.at[ix3], r3, semG)
            h1.wait()
            h2.wait()
            h3.wait()
            pltpu.sync_copy(r1, g1.at[row])
            pltpu.sync_copy(r2, g2.at[row])
            pltpu.sync_copy(r3, g3.at[row])

        def outer(i, carry):
            row = base + i
            pltpu.async_copy(d2f.at[row], bufA, semA).wait()
            process_row(row, bufA)
            return carry

        lax.fori_loop(0, rows_per, outer, 0)

    return sc_kernel


def _chain(x, corr_rep, layers, upto):
    for l in range(upto):
        w, b, gam, bet, st = layers[l]
        y = lax.dot_general(x, w, (((1,), (1,)), ((), ())),
                            precision=_HIGH) + b
        if l == 0:
            y = y - corr_rep
        mean = st[0:1, :]
        var = st[1:2, :]
        y = (y - mean) / jnp.sqrt(var + _EPS)
        y = gam * y + bet
        x = jnp.maximum(y, 0.0)
    return x


def _expand_corr(corr_blk, k):
    g0, c0 = corr_blk.shape
    e = jnp.broadcast_to(corr_blk[:, None, :], (g0, k, c0))
    return e.reshape(g0 * k, c0)


def _stats_body(k, s, rtot, g_ref, corr_ref, *refs):
    layers = []
    for l in range(s):
        w, b, gam, bet, st = refs[5 * l:5 * l + 5]
        layers.append((w[...], b[...], gam[...], bet[...], st[...]))
    w_s = refs[5 * s][...]
    b_s = refs[5 * s + 1][...]
    out_ref = refs[5 * s + 2]
    if s == 0:
        g48_ref = refs[5 * s + 3]
        acc_sum = refs[5 * s + 4]
        acc_sq = refs[5 * s + 5]
        g48 = g_ref[...][:, 0:_CP]
        g48_ref[...] = g48
    else:
        acc_sum = refs[5 * s + 3]
        acc_sq = refs[5 * s + 4]
        g48 = g_ref[...]

    corr_rep = _expand_corr(corr_ref[...], k)
    x = _chain(g48, corr_rep, layers, s)
    y = lax.dot_general(x, w_s, (((1,), (1,)), ((), ())),
                        precision=_HIGH) + b_s
    if s == 0:
        y = y - corr_rep
    c = y.shape[1]
    ys = y.reshape(_RT // 8, 8, c)
    ps = jnp.sum(ys, axis=0)
    psq = jnp.sum(ys * ys, axis=0)
    step = pl.program_id(0)

    @pl.when(step == 0)
    def _():
        acc_sum[...] = ps
        acc_sq[...] = psq

    @pl.when(step > 0)
    def _():
        acc_sum[...] = acc_sum[...] + ps
        acc_sq[...] = acc_sq[...] + psq

    @pl.when(step == pl.num_programs(0) - 1)
    def _():
        tot = jnp.sum(acc_sum[...], axis=0, keepdims=True)
        totsq = jnp.sum(acc_sq[...], axis=0, keepdims=True)
        mean = tot / rtot
        var = totsq / rtot - mean * mean
        out_ref[...] = jnp.concatenate([mean, var], axis=0)


def _final_body(k, g_ref, corr_ref, *refs):
    layers = []
    for l in range(3):
        w, b, gam, bet, st = refs[5 * l:5 * l + 5]
        layers.append((w[...], b[...], gam[...], bet[...], st[...]))
    out_ref = refs[15]
    corr_rep = _expand_corr(corr_ref[...], k)
    h = _chain(g_ref[...], corr_rep, layers, 3)
    c = h.shape[1]
    out_ref[...] = jnp.max(h.reshape(_RT // k, k, c), axis=1)


def _mlp_branch(g, corr, wps, bs, gams, bets, k):
    r = g.shape[0]
    nsteps = r // _RT
    c0 = wps[0].shape[0]
    g128_spec = pl.BlockSpec((_RT, 128), lambda t: (t, 0))
    g48_spec = pl.BlockSpec((_RT, _CP), lambda t: (t, 0))
    corr_spec = pl.BlockSpec((_RT // k, c0), lambda t: (t, 0))

    def full(a):
        return pl.BlockSpec(a.shape, lambda t: tuple(0 for _ in a.shape))

    stats = []
    g48 = None
    for s in range(3):
        cs = wps[s].shape[0]
        ins = [g if s == 0 else g48, corr]
        specs = [g128_spec if s == 0 else g48_spec, corr_spec]
        for l in range(s):
            for a in (wps[l], bs[l], gams[l], bets[l], stats[l]):
                ins.append(a)
                specs.append(full(a))
        ins += [wps[s], bs[s]]
        specs += [full(wps[s]), full(bs[s])]
        if s == 0:
            out_specs = (pl.BlockSpec((2, cs), lambda t: (0, 0)), g48_spec)
            out_shape = (jax.ShapeDtypeStruct((2, cs), jnp.float32),
                         jax.ShapeDtypeStruct((r, _CP), jnp.float32))
        else:
            out_specs = pl.BlockSpec((2, cs), lambda t: (0, 0))
            out_shape = jax.ShapeDtypeStruct((2, cs), jnp.float32)
        st = pl.pallas_call(
            functools.partial(_stats_body, k, s, float(r)),
            grid=(nsteps,),
            in_specs=specs,
            out_specs=out_specs,
            out_shape=out_shape,
            scratch_shapes=[
                pltpu.VMEM((8, cs), jnp.float32),
                pltpu.VMEM((8, cs), jnp.float32),
            ],
        )(*ins)
        if s == 0:
            st, g48 = st
        stats.append(st)

    c_last = wps[2].shape[0]
    ins = [g48, corr]
    specs = [g48_spec, corr_spec]
    for l in range(3):
        for a in (wps[l], bs[l], gams[l], bets[l], stats[l]):
            ins.append(a)
            specs.append(full(a))
    out = pl.pallas_call(
        functools.partial(_final_body, k),
        grid=(nsteps,),
        in_specs=specs,
        out_specs=pl.BlockSpec((_RT // k, c_last), lambda t: (t, 0)),
        out_shape=jax.ShapeDtypeStruct((_BP, c_last), jnp.float32),
    )(*ins)
    return out


def _prep_branch_params(branch):
    wps, bs, gams, bets = [], [], [], []
    for (w, b, gam, bet) in branch:
        cin = w.shape[1]
        if cin == 35:
            w = jnp.pad(w, ((0, 0), (0, _CP - 35)))
        wps.append(w)
        bs.append(b[None, :])
        gams.append(gam[None, :])
        bets.append(bet[None, :])
    return wps, bs, gams, bets


_sc_cache = []


def _sc_select_gather(d2f, tf):
    if not _sc_cache:
        _sc_cache.append(_make_sc_select_gather())
    return _sc_cache[0](d2f, tf)


def kernel(xyz, features, params):
    xp = jnp.transpose(xyz, (2, 0, 1))
    css = _fps_call(xp)
    xyz_ss = jnp.transpose(css, (1, 2, 0))
    cssf = css.reshape(3, _BP)

    xpb = jnp.transpose(xyz, (0, 2, 1))
    d2 = _d2_call(xpb, xyz_ss)
    d2f = d2.reshape(_BP, _N)

    tf = jnp.concatenate(
        [features, xyz, jnp.zeros((_B, _N, 128 - 35), jnp.float32)],
        axis=-1).reshape(_B * _N, 128)

    tf3 = tf.reshape(_B, _N, 128)
    iota_n = jnp.arange(_N, dtype=jnp.int32)[None, None, :]
    gs = []
    for radius, k in zip(_RADII, _KS):
        mask = d2 < jnp.float32(radius * radius)
        cand = jnp.where(mask, iota_n, _N)
        vals, _ = lax.top_k(-cand, k)
        idx = -vals
        idx = jnp.where(idx == _N, idx[:, :, 0:1], idx)
        g = jnp.take_along_axis(tf3[:, :, None, :], idx.reshape(_B, _BP // _B * k)[:, :, None, None], axis=1)
        gs.append(g.reshape(_BP, k, 128))

    outs = []
    for i, k in enumerate(_KS):
        wps, bs, gams, bets = _prep_branch_params(params[i])
        wx = params[i][0][0][:, 32:35]
        corr = _corr_call(cssf, wx)
        g = gs[i].reshape(_BP * k, 128)
        o = _mlp_branch(g, corr, wps, bs, gams, bets, k)
        outs.append(o.reshape(_B, _P, -1))

    feat = jnp.concatenate(outs, axis=-1)
    return xyz_ss, feat
